# baseline (speedup 1.0000x reference)
import functools

import jax
import jax.numpy as jnp
from jax import lax
from jax.experimental import pallas as pl
from jax.experimental.pallas import tpu as pltpu
from jax.experimental.pallas import tpu_sc as plsc

N = 10000
E = 320000
HID = 128
EDIM = 16
NG = 64
XW = 16

NTILES = 32
EPT = E // NTILES
CHUNK = 80
NCH = EPT // CHUNK
NROW = N // 16
WTILES = 10
WROW = N // WTILES

BE = 4000
BN = 2000


def _sigmoid(x):
    return 0.5 * jnp.tanh(0.5 * x) + 0.5


def _silu(x):
    return x * _sigmoid(x)


def _gather_comb_body(w, sub, nch, t1_hbm, t2_hbm, dst3_hbm, src3_hbm,
                      out_hbm, idxd_v, idxs_v, b1a, b2a, b1b, b2b,
                      s1a, s2a, s1b, s2b):
    wid = lax.axis_index("s") * 2 + lax.axis_index("c")
    pltpu.sync_copy(dst3_hbm.at[wid], idxd_v)
    pltpu.sync_copy(src3_hbm.at[wid], idxs_v)
    nsl = w // 16

    def start(k, b1, b2, s1, s2):
        pltpu.make_async_copy(t1_hbm.at[idxd_v.at[k]], b1, s1).start()
        pltpu.make_async_copy(t2_hbm.at[idxs_v.at[k]], b2, s2).start()

    def finish(k, b1, b2, s1, s2):
        pltpu.make_async_copy(t1_hbm.at[idxd_v.at[k]], b1, s1).wait()
        pltpu.make_async_copy(t2_hbm.at[idxs_v.at[k]], b2, s2).wait()

        def vrow(j, carry):
            for q in range(16):
                r = j * 16 + q
                for t in range(nsl):
                    sl = pl.ds(t * 16, 16)
                    if sub:
                        b1[r, sl] = b1[r, sl] - b2[r, sl]
                    else:
                        b1[r, sl] = b1[r, sl] + b2[r, sl]
            return carry

        lax.fori_loop(0, CHUNK // 16, vrow, 0)
        pltpu.sync_copy(
            b1, out_hbm.at[pl.ds(wid * (nch * CHUNK) + k * CHUNK, CHUNK)])

    start(0, b1a, b2a, s1a, s2a)

    def body(i, carry):
        start(2 * i + 1, b1b, b2b, s1b, s2b)
        finish(2 * i, b1a, b2a, s1a, s2a)
        start(2 * i + 2, b1a, b2a, s1a, s2a)
        finish(2 * i + 1, b1b, b2b, s1b, s2b)
        return carry

    lax.fori_loop(0, (nch - 1) // 2, body, 0)
    if nch % 2 == 1:
        finish(nch - 1, b1a, b2a, s1a, s2a)
    else:
        start(nch - 1, b1b, b2b, s1b, s2b)
        finish(nch - 2, b1a, b2a, s1a, s2a)
        finish(nch - 1, b1b, b2b, s1b, s2b)


def _scatter_body(w, nw, wrow, nch_a, nch_b, va_hbm, vb_hbm, d3a_hbm,
                  d3b_hbm, zeros_hbm, p_hbm, acc_sh, idxa_v, idxb_v, v_v):
    c = lax.axis_index("c")
    s = lax.axis_index("s")
    wid = s * 2 + c
    @pl.when(s < nw)
    def _():
        pltpu.sync_copy(zeros_hbm.at[pl.ds(s * wrow, wrow)],
                        acc_sh.at[pl.ds(s * wrow, wrow)])
    plsc.subcore_barrier()
    pltpu.sync_copy(d3a_hbm.at[wid], idxa_v)
    pltpu.sync_copy(d3b_hbm.at[wid], idxb_v)

    def mk_body(v_hbm, idx_v, nch):
        def body(k, carry):
            base = wid * (nch * CHUNK) + k * CHUNK
            pltpu.sync_copy(v_hbm.at[pl.ds(base, CHUNK)], v_v)
            pltpu.sync_copy(v_v, acc_sh.at[idx_v.at[k]], add=True)
            return carry
        return body

    lax.fori_loop(0, nch_a, mk_body(va_hbm, idxa_v, nch_a), 0)
    lax.fori_loop(0, nch_b, mk_body(vb_hbm, idxb_v, nch_b), 0)
    plsc.subcore_barrier()
    @pl.when(s < nw)
    def _():
        pltpu.sync_copy(acc_sh.at[pl.ds(s * wrow, wrow)], p_hbm.at[c, s])


@functools.cache
def _sc_gathers(nch):
    mesh = plsc.VectorSubcoreMesh(core_axis_name="c", subcore_axis_name="s")
    sc_tiling = pltpu.CompilerParams(use_tc_tiling_on_sc=False)
    e_half = NTILES * nch * CHUNK

    def gather_comb(width, sub, params):
        return pl.kernel(
            functools.partial(_gather_comb_body, width, sub, nch),
            out_type=jax.ShapeDtypeStruct((e_half, width), jnp.float32),
            mesh=mesh,
            compiler_params=params,
            scratch_types=[pltpu.VMEM((nch, CHUNK), jnp.int32),
                           pltpu.VMEM((nch, CHUNK), jnp.int32),
                           pltpu.VMEM((CHUNK, width), jnp.float32),
                           pltpu.VMEM((CHUNK, width), jnp.float32),
                           pltpu.VMEM((CHUNK, width), jnp.float32),
                           pltpu.VMEM((CHUNK, width), jnp.float32),
                           pltpu.SemaphoreType.DMA,
                           pltpu.SemaphoreType.DMA,
                           pltpu.SemaphoreType.DMA,
                           pltpu.SemaphoreType.DMA],
        )

    return {
        "gather_f": gather_comb(HID, False, None),
        "gather_x": gather_comb(XW, True, sc_tiling),
    }


@functools.cache
def _sc_scatters(nch_a, nch_b):
    mesh = plsc.VectorSubcoreMesh(core_axis_name="c", subcore_axis_name="s")
    sc_tiling = pltpu.CompilerParams(use_tc_tiling_on_sc=False)

    def scatter(width, nw, wrow, params):
        return pl.kernel(
            functools.partial(_scatter_body, width, nw, wrow, nch_a, nch_b),
            out_type=jax.ShapeDtypeStruct((2, nw, wrow, width), jnp.float32),
            mesh=mesh,
            compiler_params=params,
            scratch_types=[pltpu.VMEM_SHARED((N, width), jnp.float32),
                           pltpu.VMEM((nch_a, CHUNK), jnp.int32),
                           pltpu.VMEM((nch_b, CHUNK), jnp.int32),
                           pltpu.VMEM((CHUNK, width), jnp.float32)],
        )

    return {
        "scatter_m": scatter(HID, WTILES, WROW, None),
        "scatter_x": scatter(XW, 16, NROW, sc_tiling),
    }


def _full(shape):
    return pl.BlockSpec(shape, lambda i: (0, 0))


def _rows(shape):
    return pl.BlockSpec(shape, lambda i: (i, 0))


def _dot(a, b):
    return jnp.dot(a, b, preferred_element_type=jnp.float32)


def _b(x):
    return x.astype(jnp.bfloat16).astype(jnp.float32)


def _node_init_body(na_ref, wemb, bemb, wi, bi, wj, h_ref, t1_ref, t2_ref):
    h = _dot(na_ref[...], wemb[...]) + bemb[...]
    h_ref[...] = h
    t1_ref[...] = _dot(h, wi[...]) + bi[...]
    t2_ref[...] = _dot(h, wj[...])


def _edge_body(with_dx, g_ref, d_ref, ea_ref, we, wr, we1, be1,
               winf, binf, wx0, bx0, wx1, bx1, m_ref, dx_ref=None):
    g = g_ref[...]
    d = d_ref[...]
    r2 = jnp.sum(d * d, axis=1, keepdims=True)
    pre = g + _b(r2) * _b(wr[...]) + _dot(ea_ref[...], we[...])
    u = _silu(pre)
    m1 = _silu(_dot(u, we1[...]) + be1[...])
    gate = _sigmoid(
        jnp.sum(_b(m1) * _b(winf[...]), axis=1, keepdims=True) + binf[...])
    m = gate * m1
    m_ref[...] = m
    if with_dx:
        t = _silu(_dot(m, wx0[...]) + bx0[...])
        coef = jnp.sum(_b(t) * _b(wx1[...]), axis=1, keepdims=True) + bx1[...]
        dx_ref[...] = d * coef


def _node_mid_body(h_ref, x_ref, p0_ref, p1_ref, q0_ref, q1_ref,
                   wh0h, wh0m, bh0, wh1, bh1,
                   wi, bi, wj, hn_ref, xn_ref, t1_ref, t2_ref):
    h = h_ref[...]
    magg = p0_ref[...] + p1_ref[...]
    xn_ref[...] = x_ref[...] + (q0_ref[...] + q1_ref[...])
    u = _silu(_dot(h, wh0h[...]) + _dot(magg, wh0m[...]) + bh0[...])
    hn = _dot(u, wh1[...]) + bh1[...]
    hn_ref[...] = hn
    t1_ref[...] = _dot(hn, wi[...]) + bi[...]
    t2_ref[...] = _dot(hn, wj[...])


def _node_last_body(h_ref, p0_ref, p1_ref,
                    wh0h, wh0m, bh0, wh1, bh1, hn_ref):
    h = h_ref[...]
    magg = p0_ref[...] + p1_ref[...]
    u = _silu(_dot(h, wh0h[...]) + _dot(magg, wh0m[...]) + bh0[...])
    hn_ref[...] = _dot(u, wh1[...]) + bh1[...]


def _readout_body(h_ref, b_ref, w0, b0, w1, b1, wp0, bp0, wp1, bp1,
                  sums_ref, cnts_ref, out_ref):
    i = pl.program_id(0)

    @pl.when(i == 0)
    def _():
        sums_ref[...] = jnp.zeros_like(sums_ref)
        cnts_ref[...] = jnp.zeros_like(cnts_ref)
        out_ref[...] = jnp.zeros_like(out_ref)

    t = _silu(_dot(h_ref[...], w0[...]) + b0[...])
    t = _dot(t, w1[...]) + b1[...]
    og = (b_ref[...] == lax.broadcasted_iota(jnp.int32, (BN, NG), 1)
          ).astype(jnp.float32)
    cdims = (((0,), (0,)), ((), ()))
    sums_ref[...] += lax.dot_general(og, t, cdims,
                                     preferred_element_type=jnp.float32,
                                     precision=lax.Precision.HIGHEST)
    cnts_ref[...] += lax.dot_general(og, jnp.ones((BN, HID), jnp.float32),
                                     cdims, preferred_element_type=jnp.float32,
                                     precision=lax.Precision.HIGHEST)

    @pl.when(i == pl.num_programs(0) - 1)
    def _():
        hg = sums_ref[...] / jnp.maximum(cnts_ref[...], 1.0)
        z = _silu(_dot(hg, wp0[...]) + bp0[...])
        out_ref[...] = (jnp.sum(z * wp1[...], axis=1, keepdims=True)
                        + bp1[...])


def _node_init_call(na, wemb, bemb, wi, bi, wj):
    grid = (N // BN,)
    return pl.pallas_call(
        _node_init_body,
        grid=grid,
        in_specs=[_rows((BN, HID)),
                  _full((HID, HID)), _full((1, HID)),
                  _full((HID, HID)), _full((1, HID)), _full((HID, HID))],
        out_specs=[_rows((BN, HID)), _rows((BN, HID)), _rows((BN, HID))],
        out_shape=[jax.ShapeDtypeStruct((N, HID), jnp.float32),
                   jax.ShapeDtypeStruct((N, HID), jnp.float32),
                   jax.ShapeDtypeStruct((N, HID), jnp.float32)],
    )(na, wemb, bemb, wi, bi, wj)


def _edge_call(g, d, ea, w, be, with_dx=True):
    ne = g.shape[0]
    grid = (ne // be,)
    out_specs = [_rows((be, HID))]
    out_shape = [jax.ShapeDtypeStruct((ne, HID), jnp.float32)]
    if with_dx:
        out_specs.append(_rows((be, XW)))
        out_shape.append(jax.ShapeDtypeStruct((ne, XW), jnp.float32))
    res = pl.pallas_call(
        functools.partial(_edge_body, with_dx),
        grid=grid,
        in_specs=[_rows((be, HID)), _rows((be, XW)), _rows((be, EDIM)),
                  _full((EDIM, HID)), _full((1, HID)),
                  _full((HID, HID)), _full((1, HID)),
                  _full((1, HID)), _full((1, 1)),
                  _full((HID, HID)), _full((1, HID)),
                  _full((1, HID)), _full((1, 1))],
        out_specs=out_specs,
        out_shape=out_shape,
    )(g, d, ea, w["we"], w["wr"], w["we1"], w["be1"], w["winf"],
      w["binf"], w["wx0"], w["bx0"], w["wx1"], w["bx1"])
    return res if with_dx else (res[0], None)


def _node_mid_call(h, x16, ps, qs, w, wi, bi, wj):
    grid = (N // BN,)
    return pl.pallas_call(
        _node_mid_body,
        grid=grid,
        in_specs=[_rows((BN, HID)), _rows((BN, XW))]
                 + [_rows((BN, HID))] * 2 + [_rows((BN, XW))] * 2
                 + [_full((HID, HID)), _full((HID, HID)), _full((1, HID)),
                    _full((HID, HID)), _full((1, HID)),
                    _full((HID, HID)), _full((1, HID)), _full((HID, HID))],
        out_specs=[_rows((BN, HID)), _rows((BN, XW)),
                   _rows((BN, HID)), _rows((BN, HID))],
        out_shape=[jax.ShapeDtypeStruct((N, HID), jnp.float32),
                   jax.ShapeDtypeStruct((N, XW), jnp.float32),
                   jax.ShapeDtypeStruct((N, HID), jnp.float32),
                   jax.ShapeDtypeStruct((N, HID), jnp.float32)],
    )(h, x16, *ps, *qs, w["wh0h"], w["wh0m"], w["bh0"], w["wh1"],
      w["bh1"], wi, bi, wj)


def _node_last_call(h, ps, w):
    grid = (N // BN,)
    return pl.pallas_call(
        _node_last_body,
        grid=grid,
        in_specs=[_rows((BN, HID))] + [_rows((BN, HID))] * 2
                 + [_full((HID, HID)), _full((HID, HID)), _full((1, HID)),
                    _full((HID, HID)), _full((1, HID))],
        out_specs=[_rows((BN, HID))],
        out_shape=[jax.ShapeDtypeStruct((N, HID), jnp.float32)],
    )(h, *ps, w["wh0h"], w["wh0m"], w["bh0"], w["wh1"], w["bh1"])[0]


def _readout_call(h, bids, w):
    grid = (N // BN,)
    return pl.pallas_call(
        _readout_body,
        grid=grid,
        in_specs=[_rows((BN, HID)), _rows((BN, 1)),
                  _full((HID, HID)), _full((1, HID)),
                  _full((HID, HID)), _full((1, HID)),
                  _full((HID, HID)), _full((1, HID)),
                  _full((1, HID)), _full((1, 1))],
        out_specs=[_full((NG, HID)), _full((NG, HID)), _full((NG, 1))],
        out_shape=[jax.ShapeDtypeStruct((NG, HID), jnp.float32),
                   jax.ShapeDtypeStruct((NG, HID), jnp.float32),
                   jax.ShapeDtypeStruct((NG, 1), jnp.float32)],
    )(h, bids, w["w0"], w["b0"], w["w1"], w["b1"],
      w["wp0"], w["bp0"], w["wp1"], w["bp1"])[2]


def _row(v):
    return v.reshape(1, -1)


def kernel(node_attr, pos, edge_attr, params, edge_index, batch):
    src = edge_index[0]
    dst = edge_index[1]
    x16 = jnp.pad(pos, ((0, 0), (0, XW - 3)))
    dst3 = dst.reshape(NTILES, NCH, CHUNK)
    src3 = src.reshape(NTILES, NCH, CHUNK)
    ea4 = edge_attr.reshape(NTILES, NCH, CHUNK, EDIM)
    NA = 62
    halves = []
    for lo, hi, be in ((0, NA, 3968), (NA, NCH, 4032)):
        nch = hi - lo
        halves.append({
            "nch": nch, "be": be,
            "dst3": dst3[:, lo:hi],
            "src3": src3[:, lo:hi],
            "ea": ea4[:, lo:hi].reshape(NTILES * nch * CHUNK, EDIM),
        })
    zeros_m = jnp.zeros((N, HID), jnp.float32)
    zeros_x = jnp.zeros((N, XW), jnp.float32)
    bids = batch.reshape(N, 1)

    layers = params["layers"]

    def e0_split(lp):
        w = lp["e0"]["w"]
        return (w[:HID], _row(lp["e0"]["b"]), w[HID:2 * HID],
                w[2 * HID:2 * HID + 1], w[2 * HID + 1:])

    def edge_w(lp):
        _, _, _, wr, we = e0_split(lp)
        return {
            "we": we, "wr": wr,
            "we1": lp["e1"]["w"], "be1": _row(lp["e1"]["b"]),
            "winf": _row(lp["inf"]["w"][:, 0]),
            "binf": lp["inf"]["b"].reshape(1, 1),
            "wx0": lp["x0"]["w"], "bx0": _row(lp["x0"]["b"]),
            "wx1": _row(lp["x1"]["w"][:, 0]),
            "bx1": lp["x1"]["b"].reshape(1, 1),
        }

    def node_w(lp):
        return {
            "wh0h": lp["h0"]["w"][:HID], "wh0m": lp["h0"]["w"][HID:],
            "bh0": _row(lp["h0"]["b"]),
            "wh1": lp["h1"]["w"], "bh1": _row(lp["h1"]["b"]),
        }

    wi0, bi0, wj0, _, _ = e0_split(layers[0])
    h, t1, t2 = _node_init_call(node_attr, params["embedding"]["w"],
                                _row(params["embedding"]["b"]), wi0, bi0, wj0)

    hva, hvb = halves
    scat = _sc_scatters(hva["nch"], hvb["nch"])
    for l in range(len(layers)):
        lp = layers[l]
        last = l + 1 == len(layers)
        ew = edge_w(lp)
        mdx = []
        for hv in halves:
            gk = _sc_gathers(hv["nch"])
            g = gk["gather_f"](t1, t2, hv["dst3"], hv["src3"])
            d = gk["gather_x"](x16, x16, hv["dst3"], hv["src3"])
            mdx.append(_edge_call(g, d, hv["ea"], ew, hv["be"],
                                  with_dx=not last))
        p = scat["scatter_m"](mdx[0][0], mdx[1][0], hva["dst3"],
                              hvb["dst3"], zeros_m).reshape(2, N, HID)
        ps = [p[0], p[1]]
        if not last:
            q = scat["scatter_x"](mdx[0][1], mdx[1][1], hva["dst3"],
                                  hvb["dst3"], zeros_x).reshape(2, N, XW)
            qs = [q[0], q[1]]
            win, bin_, wjn, _, _ = e0_split(layers[l + 1])
            h, x16, t1, t2 = _node_mid_call(h, x16, ps, qs,
                                            node_w(lp), win, bin_, wjn)
        else:
            h = _node_last_call(h, ps, node_w(lp))

    rw = {
        "w0": params["lin0"]["w"], "b0": _row(params["lin0"]["b"]),
        "w1": params["lin1"]["w"], "b1": _row(params["lin1"]["b"]),
        "wp0": params["pred0"]["w"], "bp0": _row(params["pred0"]["b"]),
        "wp1": _row(params["pred1"]["w"][:, 0]),
        "bp1": params["pred1"]["b"].reshape(1, 1),
    }
    out = _readout_call(h, bids, rw)
    return out.reshape(-1)


# speedup vs baseline: 1.0135x; 1.0135x over previous
import functools

import jax
import jax.numpy as jnp
from jax import lax
from jax.experimental import pallas as pl
from jax.experimental.pallas import tpu as pltpu
from jax.experimental.pallas import tpu_sc as plsc

N = 10000
E = 320000
HID = 128
EDIM = 16
NG = 64
XW = 16

NTILES = 32
EPT = E // NTILES
CHUNK = 80
NCH = EPT // CHUNK
NROW = N // 16
WTILES = 10
WROW = N // WTILES

BE = 4000
BN = 2000


def _sigmoid(x):
    return 0.5 * jnp.tanh(0.5 * x) + 0.5


def _silu(x):
    return x * _sigmoid(x)


def _gather_comb_body(w, sub, nch, t1_hbm, t2_hbm, dst3_hbm, src3_hbm,
                      out_hbm, idxd_v, idxs_v, b1a, b2a, b1b, b2b,
                      s1a, s2a, s1b, s2b):
    wid = lax.axis_index("s") * 2 + lax.axis_index("c")
    pltpu.sync_copy(dst3_hbm.at[wid], idxd_v)
    pltpu.sync_copy(src3_hbm.at[wid], idxs_v)
    nsl = w // 16

    def start(k, b1, b2, s1, s2):
        pltpu.make_async_copy(t1_hbm.at[idxd_v.at[k]], b1, s1).start()
        pltpu.make_async_copy(t2_hbm.at[idxs_v.at[k]], b2, s2).start()

    def finish(k, b1, b2, s1, s2):
        pltpu.make_async_copy(t1_hbm.at[idxd_v.at[k]], b1, s1).wait()
        pltpu.make_async_copy(t2_hbm.at[idxs_v.at[k]], b2, s2).wait()

        def vrow(j, carry):
            for q in range(16):
                r = j * 16 + q
                for t in range(nsl):
                    sl = pl.ds(t * 16, 16)
                    if sub:
                        b1[r, sl] = b1[r, sl] - b2[r, sl]
                    else:
                        b1[r, sl] = b1[r, sl] + b2[r, sl]
            return carry

        lax.fori_loop(0, CHUNK // 16, vrow, 0)
        pltpu.sync_copy(
            b1, out_hbm.at[pl.ds(wid * (nch * CHUNK) + k * CHUNK, CHUNK)])

    start(0, b1a, b2a, s1a, s2a)

    def body(i, carry):
        start(2 * i + 1, b1b, b2b, s1b, s2b)
        finish(2 * i, b1a, b2a, s1a, s2a)
        start(2 * i + 2, b1a, b2a, s1a, s2a)
        finish(2 * i + 1, b1b, b2b, s1b, s2b)
        return carry

    lax.fori_loop(0, (nch - 1) // 2, body, 0)
    if nch % 2 == 1:
        finish(nch - 1, b1a, b2a, s1a, s2a)
    else:
        start(nch - 1, b1b, b2b, s1b, s2b)
        finish(nch - 2, b1a, b2a, s1a, s2a)
        finish(nch - 1, b1b, b2b, s1b, s2b)


def _scatter_body(w, nw, wrow, nch, v_hbm, dst3_hbm, zeros_hbm, p_hbm,
                  acc_sh, idx_v, v_v):
    c = lax.axis_index("c")
    s = lax.axis_index("s")
    wid = s * 2 + c
    @pl.when(s < nw)
    def _():
        pltpu.sync_copy(zeros_hbm.at[pl.ds(s * wrow, wrow)],
                        acc_sh.at[pl.ds(s * wrow, wrow)])
    plsc.subcore_barrier()
    pltpu.sync_copy(dst3_hbm.at[wid], idx_v)

    def body(k, carry):
        base = wid * (nch * CHUNK) + k * CHUNK
        pltpu.sync_copy(v_hbm.at[pl.ds(base, CHUNK)], v_v)
        pltpu.sync_copy(v_v, acc_sh.at[idx_v.at[k]], add=True)
        return carry

    lax.fori_loop(0, nch, body, 0)
    plsc.subcore_barrier()
    @pl.when(s < nw)
    def _():
        pltpu.sync_copy(acc_sh.at[pl.ds(s * wrow, wrow)], p_hbm.at[c, s])


@functools.cache
def _sc_gathers(nch):
    mesh = plsc.VectorSubcoreMesh(core_axis_name="c", subcore_axis_name="s")
    sc_tiling = pltpu.CompilerParams(use_tc_tiling_on_sc=False)
    e_half = NTILES * nch * CHUNK

    def gather_comb(width, sub, params):
        return pl.kernel(
            functools.partial(_gather_comb_body, width, sub, nch),
            out_type=jax.ShapeDtypeStruct((e_half, width), jnp.float32),
            mesh=mesh,
            compiler_params=params,
            scratch_types=[pltpu.VMEM((nch, CHUNK), jnp.int32),
                           pltpu.VMEM((nch, CHUNK), jnp.int32),
                           pltpu.VMEM((CHUNK, width), jnp.float32),
                           pltpu.VMEM((CHUNK, width), jnp.float32),
                           pltpu.VMEM((CHUNK, width), jnp.float32),
                           pltpu.VMEM((CHUNK, width), jnp.float32),
                           pltpu.SemaphoreType.DMA,
                           pltpu.SemaphoreType.DMA,
                           pltpu.SemaphoreType.DMA,
                           pltpu.SemaphoreType.DMA],
        )

    return {
        "gather_f": gather_comb(HID, False, None),
        "gather_x": gather_comb(XW, True, sc_tiling),
    }


@functools.cache
def _sc_scatters(nch):
    mesh = plsc.VectorSubcoreMesh(core_axis_name="c", subcore_axis_name="s")
    sc_tiling = pltpu.CompilerParams(use_tc_tiling_on_sc=False)

    def scatter(width, nw, wrow, params):
        return pl.kernel(
            functools.partial(_scatter_body, width, nw, wrow, nch),
            out_type=jax.ShapeDtypeStruct((2, nw, wrow, width), jnp.float32),
            mesh=mesh,
            compiler_params=params,
            scratch_types=[pltpu.VMEM_SHARED((N, width), jnp.float32),
                           pltpu.VMEM((nch, CHUNK), jnp.int32),
                           pltpu.VMEM((CHUNK, width), jnp.float32)],
        )

    return {
        "scatter_m": scatter(HID, WTILES, WROW, None),
        "scatter_x": scatter(XW, 16, NROW, sc_tiling),
    }


def _full(shape):
    return pl.BlockSpec(shape, lambda i: (0, 0))


def _rows(shape):
    return pl.BlockSpec(shape, lambda i: (i, 0))


def _dot(a, b):
    return jnp.dot(a, b, preferred_element_type=jnp.float32)


def _b(x):
    return x.astype(jnp.bfloat16).astype(jnp.float32)


def _node_init_body(na_ref, wemb, bemb, wi, bi, wj, h_ref, t1_ref, t2_ref):
    h = _dot(na_ref[...], wemb[...]) + bemb[...]
    h_ref[...] = h
    t1_ref[...] = _dot(h, wi[...]) + bi[...]
    t2_ref[...] = _dot(h, wj[...])


def _edge_body(with_dx, g_ref, d_ref, ea_ref, we, wr, we1, be1,
               winf, binf, wx0, bx0, wx1, bx1, m_ref, dx_ref=None):
    g = g_ref[...]
    d = d_ref[...]
    r2 = jnp.sum(d * d, axis=1, keepdims=True)
    pre = g + _b(r2) * _b(wr[...]) + _dot(ea_ref[...], we[...])
    u = _silu(pre)
    m1 = _silu(_dot(u, we1[...]) + be1[...])
    gate = _sigmoid(
        jnp.sum(_b(m1) * _b(winf[...]), axis=1, keepdims=True) + binf[...])
    m = gate * m1
    m_ref[...] = m
    if with_dx:
        t = _silu(_dot(m, wx0[...]) + bx0[...])
        coef = jnp.sum(_b(t) * _b(wx1[...]), axis=1, keepdims=True) + bx1[...]
        dx_ref[...] = d * coef


def _node_mid_body(h_ref, x_ref, p0_ref, p1_ref, p2_ref, p3_ref,
                   q0_ref, q1_ref, q2_ref, q3_ref,
                   wh0h, wh0m, bh0, wh1, bh1,
                   wi, bi, wj, hn_ref, xn_ref, t1_ref, t2_ref):
    h = h_ref[...]
    magg = (p0_ref[...] + p1_ref[...]) + (p2_ref[...] + p3_ref[...])
    xn_ref[...] = x_ref[...] + ((q0_ref[...] + q1_ref[...])
                                + (q2_ref[...] + q3_ref[...]))
    u = _silu(_dot(h, wh0h[...]) + _dot(magg, wh0m[...]) + bh0[...])
    hn = _dot(u, wh1[...]) + bh1[...]
    hn_ref[...] = hn
    t1_ref[...] = _dot(hn, wi[...]) + bi[...]
    t2_ref[...] = _dot(hn, wj[...])


def _node_last_body(h_ref, p0_ref, p1_ref, p2_ref, p3_ref,
                    wh0h, wh0m, bh0, wh1, bh1, hn_ref):
    h = h_ref[...]
    magg = (p0_ref[...] + p1_ref[...]) + (p2_ref[...] + p3_ref[...])
    u = _silu(_dot(h, wh0h[...]) + _dot(magg, wh0m[...]) + bh0[...])
    hn_ref[...] = _dot(u, wh1[...]) + bh1[...]


def _readout_body(h_ref, b_ref, w0, b0, w1, b1, wp0, bp0, wp1, bp1,
                  sums_ref, cnts_ref, out_ref):
    i = pl.program_id(0)

    @pl.when(i == 0)
    def _():
        sums_ref[...] = jnp.zeros_like(sums_ref)
        cnts_ref[...] = jnp.zeros_like(cnts_ref)
        out_ref[...] = jnp.zeros_like(out_ref)

    t = _silu(_dot(h_ref[...], w0[...]) + b0[...])
    t = _dot(t, w1[...]) + b1[...]
    og = (b_ref[...] == lax.broadcasted_iota(jnp.int32, (BN, NG), 1)
          ).astype(jnp.float32)
    cdims = (((0,), (0,)), ((), ()))
    sums_ref[...] += lax.dot_general(og, t, cdims,
                                     preferred_element_type=jnp.float32,
                                     precision=lax.Precision.HIGHEST)
    cnts_ref[...] += lax.dot_general(og, jnp.ones((BN, HID), jnp.float32),
                                     cdims, preferred_element_type=jnp.float32,
                                     precision=lax.Precision.HIGHEST)

    @pl.when(i == pl.num_programs(0) - 1)
    def _():
        hg = sums_ref[...] / jnp.maximum(cnts_ref[...], 1.0)
        z = _silu(_dot(hg, wp0[...]) + bp0[...])
        out_ref[...] = (jnp.sum(z * wp1[...], axis=1, keepdims=True)
                        + bp1[...])


def _node_init_call(na, wemb, bemb, wi, bi, wj):
    grid = (N // BN,)
    return pl.pallas_call(
        _node_init_body,
        grid=grid,
        in_specs=[_rows((BN, HID)),
                  _full((HID, HID)), _full((1, HID)),
                  _full((HID, HID)), _full((1, HID)), _full((HID, HID))],
        out_specs=[_rows((BN, HID)), _rows((BN, HID)), _rows((BN, HID))],
        out_shape=[jax.ShapeDtypeStruct((N, HID), jnp.float32),
                   jax.ShapeDtypeStruct((N, HID), jnp.float32),
                   jax.ShapeDtypeStruct((N, HID), jnp.float32)],
    )(na, wemb, bemb, wi, bi, wj)


def _edge_call(g, d, ea, w, be, with_dx=True):
    ne = g.shape[0]
    grid = (ne // be,)
    out_specs = [_rows((be, HID))]
    out_shape = [jax.ShapeDtypeStruct((ne, HID), jnp.float32)]
    if with_dx:
        out_specs.append(_rows((be, XW)))
        out_shape.append(jax.ShapeDtypeStruct((ne, XW), jnp.float32))
    res = pl.pallas_call(
        functools.partial(_edge_body, with_dx),
        grid=grid,
        in_specs=[_rows((be, HID)), _rows((be, XW)), _rows((be, EDIM)),
                  _full((EDIM, HID)), _full((1, HID)),
                  _full((HID, HID)), _full((1, HID)),
                  _full((1, HID)), _full((1, 1)),
                  _full((HID, HID)), _full((1, HID)),
                  _full((1, HID)), _full((1, 1))],
        out_specs=out_specs,
        out_shape=out_shape,
    )(g, d, ea, w["we"], w["wr"], w["we1"], w["be1"], w["winf"],
      w["binf"], w["wx0"], w["bx0"], w["wx1"], w["bx1"])
    return res if with_dx else (res[0], None)


def _node_mid_call(h, x16, ps, qs, w, wi, bi, wj):
    grid = (N // BN,)
    return pl.pallas_call(
        _node_mid_body,
        grid=grid,
        in_specs=[_rows((BN, HID)), _rows((BN, XW))]
                 + [_rows((BN, HID))] * 4 + [_rows((BN, XW))] * 4
                 + [_full((HID, HID)), _full((HID, HID)), _full((1, HID)),
                    _full((HID, HID)), _full((1, HID)),
                    _full((HID, HID)), _full((1, HID)), _full((HID, HID))],
        out_specs=[_rows((BN, HID)), _rows((BN, XW)),
                   _rows((BN, HID)), _rows((BN, HID))],
        out_shape=[jax.ShapeDtypeStruct((N, HID), jnp.float32),
                   jax.ShapeDtypeStruct((N, XW), jnp.float32),
                   jax.ShapeDtypeStruct((N, HID), jnp.float32),
                   jax.ShapeDtypeStruct((N, HID), jnp.float32)],
    )(h, x16, *ps, *qs, w["wh0h"], w["wh0m"], w["bh0"], w["wh1"],
      w["bh1"], wi, bi, wj)


def _node_last_call(h, ps, w):
    grid = (N // BN,)
    return pl.pallas_call(
        _node_last_body,
        grid=grid,
        in_specs=[_rows((BN, HID))] + [_rows((BN, HID))] * 4
                 + [_full((HID, HID)), _full((HID, HID)), _full((1, HID)),
                    _full((HID, HID)), _full((1, HID))],
        out_specs=[_rows((BN, HID))],
        out_shape=[jax.ShapeDtypeStruct((N, HID), jnp.float32)],
    )(h, *ps, w["wh0h"], w["wh0m"], w["bh0"], w["wh1"], w["bh1"])[0]


def _readout_call(h, bids, w):
    grid = (N // BN,)
    return pl.pallas_call(
        _readout_body,
        grid=grid,
        in_specs=[_rows((BN, HID)), _rows((BN, 1)),
                  _full((HID, HID)), _full((1, HID)),
                  _full((HID, HID)), _full((1, HID)),
                  _full((HID, HID)), _full((1, HID)),
                  _full((1, HID)), _full((1, 1))],
        out_specs=[_full((NG, HID)), _full((NG, HID)), _full((NG, 1))],
        out_shape=[jax.ShapeDtypeStruct((NG, HID), jnp.float32),
                   jax.ShapeDtypeStruct((NG, HID), jnp.float32),
                   jax.ShapeDtypeStruct((NG, 1), jnp.float32)],
    )(h, bids, w["w0"], w["b0"], w["w1"], w["b1"],
      w["wp0"], w["bp0"], w["wp1"], w["bp1"])[2]


def _row(v):
    return v.reshape(1, -1)


def kernel(node_attr, pos, edge_attr, params, edge_index, batch):
    src = edge_index[0]
    dst = edge_index[1]
    x16 = jnp.pad(pos, ((0, 0), (0, XW - 3)))
    dst3 = dst.reshape(NTILES, NCH, CHUNK)
    src3 = src.reshape(NTILES, NCH, CHUNK)
    ea4 = edge_attr.reshape(NTILES, NCH, CHUNK, EDIM)
    NA = 62
    halves = []
    for lo, hi, be in ((0, NA, 3968), (NA, NCH, 4032)):
        nch = hi - lo
        halves.append({
            "nch": nch, "be": be,
            "dst3": dst3[:, lo:hi],
            "src3": src3[:, lo:hi],
            "ea": ea4[:, lo:hi].reshape(NTILES * nch * CHUNK, EDIM),
        })
    zeros_m = jnp.zeros((N, HID), jnp.float32)
    zeros_x = jnp.zeros((N, XW), jnp.float32)
    bids = batch.reshape(N, 1)

    layers = params["layers"]

    def e0_split(lp):
        w = lp["e0"]["w"]
        return (w[:HID], _row(lp["e0"]["b"]), w[HID:2 * HID],
                w[2 * HID:2 * HID + 1], w[2 * HID + 1:])

    def edge_w(lp):
        _, _, _, wr, we = e0_split(lp)
        return {
            "we": we, "wr": wr,
            "we1": lp["e1"]["w"], "be1": _row(lp["e1"]["b"]),
            "winf": _row(lp["inf"]["w"][:, 0]),
            "binf": lp["inf"]["b"].reshape(1, 1),
            "wx0": lp["x0"]["w"], "bx0": _row(lp["x0"]["b"]),
            "wx1": _row(lp["x1"]["w"][:, 0]),
            "bx1": lp["x1"]["b"].reshape(1, 1),
        }

    def node_w(lp):
        return {
            "wh0h": lp["h0"]["w"][:HID], "wh0m": lp["h0"]["w"][HID:],
            "bh0": _row(lp["h0"]["b"]),
            "wh1": lp["h1"]["w"], "bh1": _row(lp["h1"]["b"]),
        }

    wi0, bi0, wj0, _, _ = e0_split(layers[0])
    h, t1, t2 = _node_init_call(node_attr, params["embedding"]["w"],
                                _row(params["embedding"]["b"]), wi0, bi0, wj0)

    for l in range(len(layers)):
        lp = layers[l]
        last = l + 1 == len(layers)
        ew = edge_w(lp)
        mdx = []
        for hv in halves:
            gk = _sc_gathers(hv["nch"])
            g = gk["gather_f"](t1, t2, hv["dst3"], hv["src3"])
            d = gk["gather_x"](x16, x16, hv["dst3"], hv["src3"])
            mdx.append(_edge_call(g, d, hv["ea"], ew, hv["be"],
                                  with_dx=not last))
        ps, qs = [], []
        for hv, (m, dx) in zip(halves, mdx):
            scat = _sc_scatters(hv["nch"])
            p = scat["scatter_m"](m, hv["dst3"], zeros_m).reshape(2, N, HID)
            ps += [p[0], p[1]]
            if not last:
                q = scat["scatter_x"](dx, hv["dst3"],
                                      zeros_x).reshape(2, N, XW)
                qs += [q[0], q[1]]
        if not last:
            win, bin_, wjn, _, _ = e0_split(layers[l + 1])
            h, x16, t1, t2 = _node_mid_call(h, x16, ps, qs,
                                            node_w(lp), win, bin_, wjn)
        else:
            h = _node_last_call(h, ps, node_w(lp))

    rw = {
        "w0": params["lin0"]["w"], "b0": _row(params["lin0"]["b"]),
        "w1": params["lin1"]["w"], "b1": _row(params["lin1"]["b"]),
        "wp0": params["pred0"]["w"], "bp0": _row(params["pred0"]["b"]),
        "wp1": _row(params["pred1"]["w"][:, 0]),
        "bp1": params["pred1"]["b"].reshape(1, 1),
    }
    out = _readout_call(h, bids, rw)
    return out.reshape(-1)
